# Initial kernel scaffold; baseline (speedup 1.0000x reference)
import jax
import jax.numpy as jnp
from jax.experimental import pallas as pl


def kernel(x, edge_index, W1, b1, W2, b2):
    raise NotImplementedError("write your pallas kernel here")


# speedup vs baseline: 22.4393x; 22.4393x over previous
import functools

import jax
import jax.numpy as jnp
from jax import lax
from jax.experimental import pallas as pl
from jax.experimental.pallas import tpu as pltpu
from jax.experimental.pallas import tpu_sc as plsc

NC = 2
NS = 16
GRP = 128


def _mesh():
  return plsc.VectorSubcoreMesh(core_axis_name="c", subcore_axis_name="s")


_SC_PARAMS = pltpu.CompilerParams(use_tc_tiling_on_sc=False)


@functools.lru_cache(maxsize=None)
def _make_deg_kernel(n_pad, n_groups):
  gpw = n_groups // (NC * NS)
  rpt = n_pad // NS

  @functools.partial(
      pl.kernel, mesh=_mesh(),
      out_type=jax.ShapeDtypeStruct((NC, n_pad), jnp.float32),
      compiler_params=_SC_PARAMS,
      scratch_types=[
          pltpu.VMEM((gpw, GRP), jnp.int32),
          pltpu.VMEM((GRP,), jnp.float32),
          pltpu.VMEM_SHARED((n_pad,), jnp.float32),
      ],
  )
  def deg_kernel(row_hbm, zeros_hbm, out_hbm, rowv, ones_v, acc):
    cid = lax.axis_index("c")
    sid = lax.axis_index("s")
    wid = sid * NC + cid
    for i in range(GRP // 16):
      ones_v[pl.ds(i * 16, 16)] = jnp.ones((16,), jnp.float32)
    pltpu.sync_copy(zeros_hbm.at[pl.ds(sid * rpt, rpt)],
                    acc.at[pl.ds(sid * rpt, rpt)])
    plsc.subcore_barrier()
    pltpu.sync_copy(row_hbm.at[pl.ds(wid * gpw, gpw)], rowv)

    def body(j, carry):
      pltpu.sync_copy(ones_v, acc.at[rowv.at[j]], add=True)
      return carry

    lax.fori_loop(0, gpw, body, 0)
    plsc.subcore_barrier()
    pltpu.sync_copy(acc.at[pl.ds(sid * rpt, rpt)],
                    out_hbm.at[cid, pl.ds(sid * rpt, rpt)])

  return deg_kernel


@functools.lru_cache(maxsize=None)
def _make_gs_kernel(n_pad, n_groups, width):
  gpw = n_groups // (NC * NS)
  rpt = n_pad // NS

  @functools.partial(
      pl.kernel, mesh=_mesh(),
      out_type=jax.ShapeDtypeStruct((NC, n_pad, width), jnp.float32),
      compiler_params=_SC_PARAMS,
      scratch_types=[
          pltpu.VMEM((gpw, GRP), jnp.int32),
          pltpu.VMEM((gpw, GRP), jnp.int32),
          pltpu.VMEM((GRP, width), jnp.float32),
          pltpu.VMEM_SHARED((n_pad, width), jnp.float32),
          pltpu.SemaphoreType.DMA,
      ],
  )
  def gs_kernel(z_hbm, col_hbm, row_hbm, zeros_hbm, out_hbm,
                colv, rowv, buf, acc, sem):
    cid = lax.axis_index("c")
    sid = lax.axis_index("s")
    wid = sid * NC + cid
    pltpu.sync_copy(zeros_hbm.at[pl.ds(sid * rpt, rpt)],
                    acc.at[pl.ds(sid * rpt, rpt)])
    plsc.subcore_barrier()
    pltpu.sync_copy(col_hbm.at[pl.ds(wid * gpw, gpw)], colv)
    pltpu.sync_copy(row_hbm.at[pl.ds(wid * gpw, gpw)], rowv)

    def body(j, carry):
      pltpu.async_copy(z_hbm.at[colv.at[j]], buf, sem).wait()
      pltpu.sync_copy(buf, acc.at[rowv.at[j]], add=True)
      return carry

    lax.fori_loop(0, gpw, body, 0)
    plsc.subcore_barrier()
    pltpu.sync_copy(acc.at[pl.ds(sid * rpt, rpt)],
                    out_hbm.at[cid, pl.ds(sid * rpt, rpt)])

  return gs_kernel


def _tc_a_body(n, deg_ref, x_ref, w1_ref, b1_ref,
               dinv_ref, hlin0_ref, z1p_ref):
  deg = deg_ref[:, 0:1] + deg_ref[:, 1:2]
  dinv = jnp.where(deg > 0, lax.rsqrt(jnp.maximum(deg, 1.0)), 0.0)
  dinv_ref[...] = dinv
  x = x_ref[...]
  hlin0_ref[...] = (
      jnp.dot(x, w1_ref[0], preferred_element_type=jnp.float32) + b1_ref[...])
  y1 = jnp.dot(x, w1_ref[1], preferred_element_type=jnp.float32)
  z1p_ref[...] = jnp.zeros_like(z1p_ref)
  z1p_ref[0:n, :] = dinv[0:n, :] * y1


def _tc_b_body(n, hlin0_ref, g1_ref, dinv_ref, w2_ref, b2_ref,
               hlin20_ref, z2p_ref):
  dinv = dinv_ref[0:n, :]
  s = g1_ref[0, 0:n, :] + g1_ref[1, 0:n, :]
  h = jnp.maximum(hlin0_ref[...] - dinv * s, 0.0)
  hlin20_ref[...] = (
      jnp.dot(h, w2_ref[0], preferred_element_type=jnp.float32) + b2_ref[...])
  y2 = jnp.dot(h, w2_ref[1], preferred_element_type=jnp.float32)
  z2p_ref[...] = jnp.zeros_like(z2p_ref)
  z2p_ref[0:n, :] = dinv * y2


def _tc_c_body(n, hlin20_ref, g2_ref, dinv_ref, out_ref):
  dinv = dinv_ref[0:n, :]
  s = g2_ref[0, 0:n, :] + g2_ref[1, 0:n, :]
  o = hlin20_ref[...] - dinv * s
  m = jnp.max(o, axis=1, keepdims=True)
  lse = jnp.log(jnp.sum(jnp.exp(o - m), axis=1, keepdims=True)) + m
  out_ref[...] = o - lse


def kernel(x, edge_index, W1, b1, W2, b2):
  n, f_in = x.shape
  e = edge_index.shape[1]
  hid = W1.shape[2]
  ncls = W2.shape[2]

  n_pad = -(-(n + 1) // (NS * GRP)) * (NS * GRP)
  e_pad = -(-e // (NC * NS * GRP * 8)) * (NC * NS * GRP * 8)
  n_groups = e_pad // GRP

  pad = jnp.full((e_pad - e,), n, dtype=jnp.int32)
  row2d = jnp.concatenate([edge_index[0], pad]).reshape(n_groups, GRP)
  col2d = jnp.concatenate([edge_index[1], pad]).reshape(n_groups, GRP)
  zeros1 = jnp.zeros((n_pad,), jnp.float32)
  zeros_h = jnp.zeros((n_pad, hid), jnp.float32)
  zeros_c = jnp.zeros((n_pad, ncls), jnp.float32)

  deg2 = _make_deg_kernel(n_pad, n_groups)(row2d, zeros1)
  degT = deg2.T

  dinvp, hlin0, z1p = pl.pallas_call(
      functools.partial(_tc_a_body, n),
      out_shape=[
          jax.ShapeDtypeStruct((n_pad, 1), jnp.float32),
          jax.ShapeDtypeStruct((n, hid), jnp.float32),
          jax.ShapeDtypeStruct((n_pad, hid), jnp.float32),
      ],
  )(degT, x, W1, b1.reshape(1, -1))

  g1 = _make_gs_kernel(n_pad, n_groups, hid)(z1p, col2d, row2d, zeros_h)

  hlin20, z2p = pl.pallas_call(
      functools.partial(_tc_b_body, n),
      out_shape=[
          jax.ShapeDtypeStruct((n, ncls), jnp.float32),
          jax.ShapeDtypeStruct((n_pad, ncls), jnp.float32),
      ],
  )(hlin0, g1, dinvp, W2, b2.reshape(1, -1))

  g2 = _make_gs_kernel(n_pad, n_groups, ncls)(z2p, col2d, row2d, zeros_c)

  out = pl.pallas_call(
      functools.partial(_tc_c_body, n),
      out_shape=jax.ShapeDtypeStruct((n, ncls), jnp.float32),
  )(hlin20, g2, dinvp)

  return out


# speedup vs baseline: 27.7964x; 1.2387x over previous
import functools

import jax
import jax.numpy as jnp
from jax import lax
from jax.experimental import pallas as pl
from jax.experimental.pallas import tpu as pltpu
from jax.experimental.pallas import tpu_sc as plsc

NC = 2
NS = 16
GRP = 128


def _mesh():
  return plsc.VectorSubcoreMesh(core_axis_name="c", subcore_axis_name="s")


_SC_PARAMS = pltpu.CompilerParams(use_tc_tiling_on_sc=False)


@functools.lru_cache(maxsize=None)
def _make_deg_kernel(n_pad, n_groups):
  gpw = n_groups // (NC * NS)
  rpt = n_pad // NS

  @functools.partial(
      pl.kernel, mesh=_mesh(),
      out_type=jax.ShapeDtypeStruct((NC, n_pad), jnp.float32),
      compiler_params=_SC_PARAMS,
      scratch_types=[
          pltpu.VMEM((gpw, GRP), jnp.int32),
          pltpu.VMEM((GRP,), jnp.float32),
          pltpu.VMEM_SHARED((n_pad,), jnp.float32),
      ],
  )
  def deg_kernel(row_hbm, zeros_hbm, out_hbm, rowv, ones_v, acc):
    cid = lax.axis_index("c")
    sid = lax.axis_index("s")
    wid = sid * NC + cid
    for i in range(GRP // 16):
      ones_v[pl.ds(i * 16, 16)] = jnp.ones((16,), jnp.float32)
    pltpu.sync_copy(zeros_hbm.at[pl.ds(sid * rpt, rpt)],
                    acc.at[pl.ds(sid * rpt, rpt)])
    plsc.subcore_barrier()
    pltpu.sync_copy(row_hbm.at[pl.ds(wid * gpw, gpw)], rowv)

    def body(j, carry):
      pltpu.sync_copy(ones_v, acc.at[rowv.at[j]], add=True)
      return carry

    lax.fori_loop(0, gpw, body, 0)
    plsc.subcore_barrier()
    pltpu.sync_copy(acc.at[pl.ds(sid * rpt, rpt)],
                    out_hbm.at[cid, pl.ds(sid * rpt, rpt)])

  return deg_kernel


@functools.lru_cache(maxsize=None)
def _make_gs_kernel(n_pad, n_groups, width):
  gpw = n_groups // (NC * NS)
  rpt = n_pad // NS
  K = 4
  nsteps = gpw // K

  @functools.partial(
      pl.kernel, mesh=_mesh(),
      out_type=jax.ShapeDtypeStruct((NC, n_pad, width), jnp.float32),
      compiler_params=_SC_PARAMS,
      scratch_types=[
          pltpu.VMEM((gpw, GRP), jnp.int32),
          pltpu.VMEM((gpw, GRP), jnp.int32),
          pltpu.VMEM((2, K, GRP, width), jnp.float32),
          pltpu.VMEM_SHARED((n_pad, width), jnp.float32),
          pltpu.SemaphoreType.DMA,
          pltpu.SemaphoreType.DMA,
      ],
  )
  def gs_kernel(z_hbm, col_hbm, row_hbm, zeros_hbm, out_hbm,
                colv, rowv, bufs, acc, sem_g, sem_s):
    cid = lax.axis_index("c")
    sid = lax.axis_index("s")
    wid = sid * NC + cid
    pltpu.sync_copy(zeros_hbm.at[pl.ds(sid * rpt, rpt)],
                    acc.at[pl.ds(sid * rpt, rpt)])
    pltpu.sync_copy(col_hbm.at[pl.ds(wid * gpw, gpw)], colv)
    pltpu.sync_copy(row_hbm.at[pl.ds(wid * gpw, gpw)], rowv)
    plsc.subcore_barrier()

    for k in range(K):
      pltpu.async_copy(z_hbm.at[colv.at[k]], bufs.at[0, k], sem_g)

    def body(s, carry):
      cur = lax.rem(s, 2)
      for k in range(K):
        pltpu.make_async_copy(z_hbm.at[colv.at[0]], bufs.at[cur, k],
                              sem_g).wait()

      @pl.when(s < nsteps - 1)
      def _fire_next():
        for k in range(K):
          pltpu.async_copy(z_hbm.at[colv.at[(s + 1) * K + k]],
                           bufs.at[1 - cur, k], sem_g)

      for k in range(K):
        pltpu.async_copy(bufs.at[cur, k], acc.at[rowv.at[s * K + k]],
                         sem_s, add=True)
      for k in range(K):
        pltpu.make_async_copy(bufs.at[cur, k], acc.at[rowv.at[0]],
                              sem_s).wait()
      return carry

    lax.fori_loop(0, nsteps, body, 0)
    plsc.subcore_barrier()
    pltpu.sync_copy(acc.at[pl.ds(sid * rpt, rpt)],
                    out_hbm.at[cid, pl.ds(sid * rpt, rpt)])

  return gs_kernel


def _tc_a_body(n, deg_ref, x_ref, w1_ref, b1_ref,
               dinv_ref, hlin0_ref, z1p_ref):
  deg = deg_ref[:, 0:1] + deg_ref[:, 1:2]
  dinv = jnp.where(deg > 0, lax.rsqrt(jnp.maximum(deg, 1.0)), 0.0)
  dinv_ref[...] = dinv
  x = x_ref[...]
  hlin0_ref[...] = (
      jnp.dot(x, w1_ref[0], preferred_element_type=jnp.float32) + b1_ref[...])
  y1 = jnp.dot(x, w1_ref[1], preferred_element_type=jnp.float32)
  z1p_ref[...] = jnp.zeros_like(z1p_ref)
  z1p_ref[0:n, :] = dinv[0:n, :] * y1


def _tc_b_body(n, hlin0_ref, g1_ref, dinv_ref, w2_ref, b2_ref,
               hlin20_ref, z2p_ref):
  dinv = dinv_ref[0:n, :]
  s = g1_ref[0, 0:n, :] + g1_ref[1, 0:n, :]
  h = jnp.maximum(hlin0_ref[...] - dinv * s, 0.0)
  hlin20_ref[...] = (
      jnp.dot(h, w2_ref[0], preferred_element_type=jnp.float32) + b2_ref[...])
  y2 = jnp.dot(h, w2_ref[1], preferred_element_type=jnp.float32)
  z2p_ref[...] = jnp.zeros_like(z2p_ref)
  z2p_ref[0:n, :] = dinv * y2


def _tc_c_body(n, hlin20_ref, g2_ref, dinv_ref, out_ref):
  dinv = dinv_ref[0:n, :]
  s = g2_ref[0, 0:n, :] + g2_ref[1, 0:n, :]
  o = hlin20_ref[...] - dinv * s
  m = jnp.max(o, axis=1, keepdims=True)
  lse = jnp.log(jnp.sum(jnp.exp(o - m), axis=1, keepdims=True)) + m
  out_ref[...] = o - lse


def kernel(x, edge_index, W1, b1, W2, b2):
  n, f_in = x.shape
  e = edge_index.shape[1]
  hid = W1.shape[2]
  ncls = W2.shape[2]

  n_pad = -(-(n + 1) // (NS * GRP)) * (NS * GRP)
  e_pad = -(-e // (NC * NS * GRP * 8)) * (NC * NS * GRP * 8)
  n_groups = e_pad // GRP

  pad = jnp.full((e_pad - e,), n, dtype=jnp.int32)
  row2d = jnp.concatenate([edge_index[0], pad]).reshape(n_groups, GRP)
  col2d = jnp.concatenate([edge_index[1], pad]).reshape(n_groups, GRP)
  zeros1 = jnp.zeros((n_pad,), jnp.float32)
  zeros_h = jnp.zeros((n_pad, hid), jnp.float32)
  zeros_c = jnp.zeros((n_pad, ncls), jnp.float32)

  deg2 = _make_deg_kernel(n_pad, n_groups)(row2d, zeros1)
  degT = deg2.T

  dinvp, hlin0, z1p = pl.pallas_call(
      functools.partial(_tc_a_body, n),
      out_shape=[
          jax.ShapeDtypeStruct((n_pad, 1), jnp.float32),
          jax.ShapeDtypeStruct((n, hid), jnp.float32),
          jax.ShapeDtypeStruct((n_pad, hid), jnp.float32),
      ],
  )(degT, x, W1, b1.reshape(1, -1))

  g1 = _make_gs_kernel(n_pad, n_groups, hid)(z1p, col2d, row2d, zeros_h)

  hlin20, z2p = pl.pallas_call(
      functools.partial(_tc_b_body, n),
      out_shape=[
          jax.ShapeDtypeStruct((n, ncls), jnp.float32),
          jax.ShapeDtypeStruct((n_pad, ncls), jnp.float32),
      ],
  )(hlin0, g1, dinvp, W2, b2.reshape(1, -1))

  g2 = _make_gs_kernel(n_pad, n_groups, ncls)(z2p, col2d, row2d, zeros_c)

  out = pl.pallas_call(
      functools.partial(_tc_c_body, n),
      out_shape=jax.ShapeDtypeStruct((n, ncls), jnp.float32),
  )(hlin20, g2, dinvp)

  return out


# speedup vs baseline: 43.8654x; 1.5781x over previous
import functools

import jax
import jax.numpy as jnp
from jax import lax
from jax.experimental import pallas as pl
from jax.experimental.pallas import tpu as pltpu
from jax.experimental.pallas import tpu_sc as plsc

NC = 2
NS = 16
GRP = 128


def _mesh():
  return plsc.VectorSubcoreMesh(core_axis_name="c", subcore_axis_name="s")


_SC_PARAMS = pltpu.CompilerParams(use_tc_tiling_on_sc=False)


@functools.lru_cache(maxsize=None)
def _make_deg_kernel(n_pad, n_groups):
  gpw = n_groups // (NC * NS)
  rpt = n_pad // NS

  @functools.partial(
      pl.kernel, mesh=_mesh(),
      out_type=jax.ShapeDtypeStruct((NC, n_pad), jnp.float32),
      compiler_params=_SC_PARAMS,
      scratch_types=[
          pltpu.VMEM((gpw, GRP), jnp.int32),
          pltpu.VMEM((GRP,), jnp.float32),
          pltpu.VMEM_SHARED((n_pad,), jnp.float32),
      ],
  )
  def deg_kernel(row_hbm, zeros_hbm, out_hbm, rowv, ones_v, acc):
    cid = lax.axis_index("c")
    sid = lax.axis_index("s")
    wid = sid * NC + cid
    for i in range(GRP // 16):
      ones_v[pl.ds(i * 16, 16)] = jnp.ones((16,), jnp.float32)
    pltpu.sync_copy(zeros_hbm.at[pl.ds(sid * rpt, rpt)],
                    acc.at[pl.ds(sid * rpt, rpt)])
    plsc.subcore_barrier()
    pltpu.sync_copy(row_hbm.at[pl.ds(wid * gpw, gpw)], rowv)

    def body(j, carry):
      pltpu.sync_copy(ones_v, acc.at[rowv.at[j]], add=True)
      return carry

    lax.fori_loop(0, gpw, body, 0)
    plsc.subcore_barrier()
    pltpu.sync_copy(acc.at[pl.ds(sid * rpt, rpt)],
                    out_hbm.at[cid, pl.ds(sid * rpt, rpt)])

  return deg_kernel


@functools.lru_cache(maxsize=None)
def _make_gs_kernel(n_pad, n_groups, width):
  gpw = n_groups // (NC * NS)
  rpt = n_pad // NS
  K = 4
  nsteps = gpw // K

  @functools.partial(
      pl.kernel, mesh=_mesh(),
      out_type=jax.ShapeDtypeStruct((NC, n_pad, width), jnp.float32),
      compiler_params=_SC_PARAMS,
      scratch_types=[
          pltpu.VMEM((gpw, GRP), jnp.int32),
          pltpu.VMEM((gpw, GRP), jnp.int32),
          pltpu.VMEM((2, K, GRP, width), jnp.float32),
          pltpu.VMEM_SHARED((n_pad, width), jnp.float32),
          pltpu.VMEM_SHARED((n_pad, width), jnp.float32),
          pltpu.SemaphoreType.DMA,
          pltpu.SemaphoreType.DMA,
      ],
  )
  def gs_kernel(z_hbm, col_hbm, row_hbm, zeros_hbm, out_hbm,
                colv, rowv, bufs, acc, z_sh, sem_g, sem_s):
    cid = lax.axis_index("c")
    sid = lax.axis_index("s")
    wid = sid * NC + cid
    pltpu.sync_copy(zeros_hbm.at[pl.ds(sid * rpt, rpt)],
                    acc.at[pl.ds(sid * rpt, rpt)])
    pltpu.sync_copy(z_hbm.at[pl.ds(sid * rpt, rpt)],
                    z_sh.at[pl.ds(sid * rpt, rpt)])
    pltpu.sync_copy(col_hbm.at[pl.ds(wid * gpw, gpw)], colv)
    pltpu.sync_copy(row_hbm.at[pl.ds(wid * gpw, gpw)], rowv)
    plsc.subcore_barrier()

    for k in range(K):
      pltpu.async_copy(z_sh.at[colv.at[k]], bufs.at[0, k], sem_g)

    def body(s, carry):
      cur = lax.rem(s, 2)
      for k in range(K):
        pltpu.make_async_copy(z_sh.at[colv.at[0]], bufs.at[cur, k],
                              sem_g).wait()

      @pl.when(s < nsteps - 1)
      def _fire_next():
        for k in range(K):
          pltpu.async_copy(z_sh.at[colv.at[(s + 1) * K + k]],
                           bufs.at[1 - cur, k], sem_g)

      for k in range(K):
        pltpu.async_copy(bufs.at[cur, k], acc.at[rowv.at[s * K + k]],
                         sem_s, add=True)
      for k in range(K):
        pltpu.make_async_copy(bufs.at[cur, k], acc.at[rowv.at[0]],
                              sem_s).wait()
      return carry

    lax.fori_loop(0, nsteps, body, 0)
    plsc.subcore_barrier()
    pltpu.sync_copy(acc.at[pl.ds(sid * rpt, rpt)],
                    out_hbm.at[cid, pl.ds(sid * rpt, rpt)])

  return gs_kernel


def _tc_a_body(n, deg_ref, x_ref, w1_ref, b1_ref,
               dinv_ref, hlin0_ref, z1p_ref):
  deg = deg_ref[:, 0:1] + deg_ref[:, 1:2]
  dinv = jnp.where(deg > 0, lax.rsqrt(jnp.maximum(deg, 1.0)), 0.0)
  dinv_ref[...] = dinv
  x = x_ref[...]
  hlin0_ref[...] = (
      jnp.dot(x, w1_ref[0], preferred_element_type=jnp.float32) + b1_ref[...])
  y1 = jnp.dot(x, w1_ref[1], preferred_element_type=jnp.float32)
  z1p_ref[...] = jnp.zeros_like(z1p_ref)
  z1p_ref[0:n, :] = dinv[0:n, :] * y1


def _tc_b_body(n, hlin0_ref, g1_ref, dinv_ref, w2_ref, b2_ref,
               hlin20_ref, z2p_ref):
  dinv = dinv_ref[0:n, :]
  s = g1_ref[0, 0:n, :] + g1_ref[1, 0:n, :]
  h = jnp.maximum(hlin0_ref[...] - dinv * s, 0.0)
  hlin20_ref[...] = (
      jnp.dot(h, w2_ref[0], preferred_element_type=jnp.float32) + b2_ref[...])
  y2 = jnp.dot(h, w2_ref[1], preferred_element_type=jnp.float32)
  z2p_ref[...] = jnp.zeros_like(z2p_ref)
  z2p_ref[0:n, :] = dinv * y2


def _tc_c_body(n, hlin20_ref, g2_ref, dinv_ref, out_ref):
  dinv = dinv_ref[0:n, :]
  s = g2_ref[0, 0:n, :] + g2_ref[1, 0:n, :]
  o = hlin20_ref[...] - dinv * s
  m = jnp.max(o, axis=1, keepdims=True)
  lse = jnp.log(jnp.sum(jnp.exp(o - m), axis=1, keepdims=True)) + m
  out_ref[...] = o - lse


def kernel(x, edge_index, W1, b1, W2, b2):
  n, f_in = x.shape
  e = edge_index.shape[1]
  hid = W1.shape[2]
  ncls = W2.shape[2]

  n_pad = -(-(n + 1) // (NS * GRP)) * (NS * GRP)
  e_pad = -(-e // (NC * NS * GRP * 8)) * (NC * NS * GRP * 8)
  n_groups = e_pad // GRP

  pad = jnp.full((e_pad - e,), n, dtype=jnp.int32)
  row2d = jnp.concatenate([edge_index[0], pad]).reshape(n_groups, GRP)
  col2d = jnp.concatenate([edge_index[1], pad]).reshape(n_groups, GRP)
  zeros1 = jnp.zeros((n_pad,), jnp.float32)
  zeros_h = jnp.zeros((n_pad, hid), jnp.float32)
  zeros_c = jnp.zeros((n_pad, ncls), jnp.float32)

  deg2 = _make_deg_kernel(n_pad, n_groups)(row2d, zeros1)
  degT = deg2.T

  dinvp, hlin0, z1p = pl.pallas_call(
      functools.partial(_tc_a_body, n),
      out_shape=[
          jax.ShapeDtypeStruct((n_pad, 1), jnp.float32),
          jax.ShapeDtypeStruct((n, hid), jnp.float32),
          jax.ShapeDtypeStruct((n_pad, hid), jnp.float32),
      ],
  )(degT, x, W1, b1.reshape(1, -1))

  g1 = _make_gs_kernel(n_pad, n_groups, hid)(z1p, col2d, row2d, zeros_h)

  hlin20, z2p = pl.pallas_call(
      functools.partial(_tc_b_body, n),
      out_shape=[
          jax.ShapeDtypeStruct((n, ncls), jnp.float32),
          jax.ShapeDtypeStruct((n_pad, ncls), jnp.float32),
      ],
  )(hlin0, g1, dinvp, W2, b2.reshape(1, -1))

  g2 = _make_gs_kernel(n_pad, n_groups, ncls)(z2p, col2d, row2d, zeros_c)

  out = pl.pallas_call(
      functools.partial(_tc_c_body, n),
      out_shape=jax.ShapeDtypeStruct((n, ncls), jnp.float32),
  )(hlin20, g2, dinvp)

  return out


# speedup vs baseline: 44.0592x; 1.0044x over previous
import functools

import jax
import jax.numpy as jnp
from jax import lax
from jax.experimental import pallas as pl
from jax.experimental.pallas import tpu as pltpu
from jax.experimental.pallas import tpu_sc as plsc

NC = 2
NS = 16
GRP = 128


def _mesh():
  return plsc.VectorSubcoreMesh(core_axis_name="c", subcore_axis_name="s")


_SC_PARAMS = pltpu.CompilerParams(use_tc_tiling_on_sc=False)


@functools.lru_cache(maxsize=None)
def _make_deg_kernel(n_pad, n_groups):
  gpw = n_groups // (NC * NS)
  rpt = n_pad // NS

  @functools.partial(
      pl.kernel, mesh=_mesh(),
      out_type=jax.ShapeDtypeStruct((NC, n_pad), jnp.float32),
      compiler_params=_SC_PARAMS,
      scratch_types=[
          pltpu.VMEM((gpw, GRP), jnp.int32),
          pltpu.VMEM((GRP,), jnp.float32),
          pltpu.VMEM_SHARED((n_pad,), jnp.float32),
      ],
  )
  def deg_kernel(row_hbm, zeros_hbm, out_hbm, rowv, ones_v, acc):
    cid = lax.axis_index("c")
    sid = lax.axis_index("s")
    wid = sid * NC + cid
    for i in range(GRP // 16):
      ones_v[pl.ds(i * 16, 16)] = jnp.ones((16,), jnp.float32)
    pltpu.sync_copy(zeros_hbm.at[pl.ds(sid * rpt, rpt)],
                    acc.at[pl.ds(sid * rpt, rpt)])
    plsc.subcore_barrier()
    pltpu.sync_copy(row_hbm.at[pl.ds(wid * gpw, gpw)], rowv)

    def body(j, carry):
      pltpu.sync_copy(ones_v, acc.at[rowv.at[j]], add=True)
      return carry

    lax.fori_loop(0, gpw, body, 0)
    plsc.subcore_barrier()
    pltpu.sync_copy(acc.at[pl.ds(sid * rpt, rpt)],
                    out_hbm.at[cid, pl.ds(sid * rpt, rpt)])

  return deg_kernel


@functools.lru_cache(maxsize=None)
def _make_gs_kernel(n_pad, n_groups, width):
  gpw = n_groups // (NC * NS)
  rpt = n_pad // NS
  K = 4
  nsteps = gpw // K

  @functools.partial(
      pl.kernel, mesh=_mesh(),
      out_type=jax.ShapeDtypeStruct((NC, n_pad, width), jnp.float32),
      compiler_params=_SC_PARAMS,
      scratch_types=[
          pltpu.VMEM((gpw, GRP), jnp.int32),
          pltpu.VMEM((gpw, GRP), jnp.int32),
          pltpu.VMEM((2, K, GRP, width), jnp.float32),
          pltpu.VMEM_SHARED((n_pad, width), jnp.float32),
          pltpu.VMEM_SHARED((n_pad, width), jnp.float32),
          pltpu.SemaphoreType.DMA,
          pltpu.SemaphoreType.DMA,
      ],
  )
  def gs_kernel(z_hbm, col_hbm, row_hbm, zeros_hbm, out_hbm,
                colv, rowv, bufs, acc, z_sh, sem_g, sem_s):
    cid = lax.axis_index("c")
    sid = lax.axis_index("s")
    wid = sid * NC + cid
    pltpu.sync_copy(zeros_hbm.at[pl.ds(sid * rpt, rpt)],
                    acc.at[pl.ds(sid * rpt, rpt)])
    pltpu.sync_copy(z_hbm.at[pl.ds(sid * rpt, rpt)],
                    z_sh.at[pl.ds(sid * rpt, rpt)])
    pltpu.sync_copy(col_hbm.at[pl.ds(wid * gpw, gpw)], colv)
    pltpu.sync_copy(row_hbm.at[pl.ds(wid * gpw, gpw)], rowv)
    plsc.subcore_barrier()

    for k in range(K):
      pltpu.async_copy(z_sh.at[colv.at[k]], bufs.at[0, k], sem_g)

    def body(s, carry):
      cur = lax.rem(s, 2)
      for k in range(K):
        pltpu.make_async_copy(z_sh.at[colv.at[0]], bufs.at[cur, k],
                              sem_g).wait()

      @pl.when(s < nsteps - 1)
      def _fire_next():
        for k in range(K):
          pltpu.async_copy(z_sh.at[colv.at[(s + 1) * K + k]],
                           bufs.at[1 - cur, k], sem_g)

      for k in range(K):
        pltpu.async_copy(bufs.at[cur, k], acc.at[rowv.at[s * K + k]],
                         sem_s, add=True)
      for k in range(K):
        pltpu.make_async_copy(bufs.at[cur, k], acc.at[rowv.at[0]],
                              sem_s).wait()
      return carry

    lax.fori_loop(0, nsteps, body, 0)
    plsc.subcore_barrier()
    pltpu.sync_copy(acc.at[pl.ds(sid * rpt, rpt)],
                    out_hbm.at[cid, pl.ds(sid * rpt, rpt)])

  return gs_kernel


def _tc_a0_body(x_ref, w1_ref, b1_ref, hlin0_ref, y1_ref):
  x = x_ref[...]
  hlin0_ref[...] = (
      jnp.dot(x, w1_ref[0], preferred_element_type=jnp.float32) + b1_ref[...])
  y1_ref[...] = jnp.dot(x, w1_ref[1], preferred_element_type=jnp.float32)


def _tc_a1_body(n, deg_ref, y1_ref, dinv_ref, z1p_ref):
  deg = deg_ref[:, 0:1] + deg_ref[:, 1:2]
  dinv = jnp.where(deg > 0, lax.rsqrt(jnp.maximum(deg, 1.0)), 0.0)
  dinv_ref[...] = dinv
  z1p_ref[...] = jnp.zeros_like(z1p_ref)
  z1p_ref[0:n, :] = dinv[0:n, :] * y1_ref[...]


def _tc_b_body(n, hlin0_ref, g1_ref, dinv_ref, w2_ref, b2_ref,
               hlin20_ref, z2p_ref):
  dinv = dinv_ref[0:n, :]
  s = g1_ref[0, 0:n, :] + g1_ref[1, 0:n, :]
  h = jnp.maximum(hlin0_ref[...] - dinv * s, 0.0)
  hlin20_ref[...] = (
      jnp.dot(h, w2_ref[0], preferred_element_type=jnp.float32) + b2_ref[...])
  y2 = jnp.dot(h, w2_ref[1], preferred_element_type=jnp.float32)
  z2p_ref[...] = jnp.zeros_like(z2p_ref)
  z2p_ref[0:n, :] = dinv * y2


def _tc_c_body(n, hlin20_ref, g2_ref, dinv_ref, out_ref):
  dinv = dinv_ref[0:n, :]
  s = g2_ref[0, 0:n, :] + g2_ref[1, 0:n, :]
  o = hlin20_ref[...] - dinv * s
  m = jnp.max(o, axis=1, keepdims=True)
  lse = jnp.log(jnp.sum(jnp.exp(o - m), axis=1, keepdims=True)) + m
  out_ref[...] = o - lse


def kernel(x, edge_index, W1, b1, W2, b2):
  n, f_in = x.shape
  e = edge_index.shape[1]
  hid = W1.shape[2]
  ncls = W2.shape[2]

  n_pad = -(-(n + 1) // (NS * GRP)) * (NS * GRP)
  e_pad = -(-e // (NC * NS * GRP * 8)) * (NC * NS * GRP * 8)
  n_groups = e_pad // GRP

  pad = jnp.full((e_pad - e,), n, dtype=jnp.int32)
  row2d = jnp.concatenate([edge_index[0], pad]).reshape(n_groups, GRP)
  col2d = jnp.concatenate([edge_index[1], pad]).reshape(n_groups, GRP)
  zeros1 = jnp.zeros((n_pad,), jnp.float32)
  zeros_h = jnp.zeros((n_pad, hid), jnp.float32)
  zeros_c = jnp.zeros((n_pad, ncls), jnp.float32)

  deg2 = _make_deg_kernel(n_pad, n_groups)(row2d, zeros1)
  degT = deg2.T

  hlin0, y1 = pl.pallas_call(
      _tc_a0_body,
      out_shape=[
          jax.ShapeDtypeStruct((n, hid), jnp.float32),
          jax.ShapeDtypeStruct((n, hid), jnp.float32),
      ],
  )(x, W1, b1.reshape(1, -1))

  dinvp, z1p = pl.pallas_call(
      functools.partial(_tc_a1_body, n),
      out_shape=[
          jax.ShapeDtypeStruct((n_pad, 1), jnp.float32),
          jax.ShapeDtypeStruct((n_pad, hid), jnp.float32),
      ],
  )(degT, y1)

  g1 = _make_gs_kernel(n_pad, n_groups, hid)(z1p, col2d, row2d, zeros_h)

  hlin20, z2p = pl.pallas_call(
      functools.partial(_tc_b_body, n),
      out_shape=[
          jax.ShapeDtypeStruct((n, ncls), jnp.float32),
          jax.ShapeDtypeStruct((n_pad, ncls), jnp.float32),
      ],
  )(hlin0, g1, dinvp, W2, b2.reshape(1, -1))

  g2 = _make_gs_kernel(n_pad, n_groups, ncls)(z2p, col2d, row2d, zeros_c)

  out = pl.pallas_call(
      functools.partial(_tc_c_body, n),
      out_shape=jax.ShapeDtypeStruct((n, ncls), jnp.float32),
  )(hlin20, g2, dinvp)

  return out


# speedup vs baseline: 44.5374x; 1.0109x over previous
import functools

import jax
import jax.numpy as jnp
from jax import lax
from jax.experimental import pallas as pl
from jax.experimental.pallas import tpu as pltpu
from jax.experimental.pallas import tpu_sc as plsc

NC = 2
NS = 16
GRP = 128


def _mesh():
  return plsc.VectorSubcoreMesh(core_axis_name="c", subcore_axis_name="s")


_SC_PARAMS = pltpu.CompilerParams(use_tc_tiling_on_sc=False)


@functools.lru_cache(maxsize=None)
def _make_deg_kernel(n_pad, n_groups):
  gpw = n_groups // (NC * NS)
  rpt = n_pad // NS

  @functools.partial(
      pl.kernel, mesh=_mesh(),
      out_type=jax.ShapeDtypeStruct((NC, n_pad), jnp.float32),
      compiler_params=_SC_PARAMS,
      scratch_types=[
          pltpu.VMEM((gpw, GRP), jnp.int32),
          pltpu.VMEM((GRP,), jnp.float32),
          pltpu.VMEM_SHARED((n_pad,), jnp.float32),
          pltpu.SemaphoreType.DMA,
      ],
  )
  def deg_kernel(row_hbm, zeros_hbm, out_hbm, rowv, ones_v, acc, sem):
    cid = lax.axis_index("c")
    sid = lax.axis_index("s")
    wid = sid * NC + cid
    for i in range(GRP // 16):
      ones_v[pl.ds(i * 16, 16)] = jnp.ones((16,), jnp.float32)
    pltpu.sync_copy(zeros_hbm.at[pl.ds(sid * rpt, rpt)],
                    acc.at[pl.ds(sid * rpt, rpt)])
    plsc.subcore_barrier()
    pltpu.sync_copy(row_hbm.at[pl.ds(wid * gpw, gpw)], rowv)

    DK = 8

    def body(s, carry):
      for k in range(DK):
        pltpu.async_copy(ones_v, acc.at[rowv.at[s * DK + k]], sem, add=True)
      for k in range(DK):
        pltpu.make_async_copy(ones_v, acc.at[rowv.at[0]], sem).wait()
      return carry

    lax.fori_loop(0, gpw // DK, body, 0)
    plsc.subcore_barrier()
    pltpu.sync_copy(acc.at[pl.ds(sid * rpt, rpt)],
                    out_hbm.at[cid, pl.ds(sid * rpt, rpt)])

  return deg_kernel


@functools.lru_cache(maxsize=None)
def _make_gs_kernel(n_pad, n_groups, width):
  gpw = n_groups // (NC * NS)
  rpt = n_pad // NS
  K = 8
  nsteps = gpw // K

  @functools.partial(
      pl.kernel, mesh=_mesh(),
      out_type=jax.ShapeDtypeStruct((NC, n_pad, width), jnp.float32),
      compiler_params=_SC_PARAMS,
      scratch_types=[
          pltpu.VMEM((gpw, GRP), jnp.int32),
          pltpu.VMEM((gpw, GRP), jnp.int32),
          pltpu.VMEM((2, K, GRP, width), jnp.float32),
          pltpu.VMEM_SHARED((n_pad, width), jnp.float32),
          pltpu.VMEM_SHARED((n_pad, width), jnp.float32),
          pltpu.SemaphoreType.DMA,
          pltpu.SemaphoreType.DMA,
      ],
  )
  def gs_kernel(z_hbm, col_hbm, row_hbm, zeros_hbm, out_hbm,
                colv, rowv, bufs, acc, z_sh, sem_g, sem_s):
    cid = lax.axis_index("c")
    sid = lax.axis_index("s")
    wid = sid * NC + cid
    pltpu.sync_copy(zeros_hbm.at[pl.ds(sid * rpt, rpt)],
                    acc.at[pl.ds(sid * rpt, rpt)])
    pltpu.sync_copy(z_hbm.at[pl.ds(sid * rpt, rpt)],
                    z_sh.at[pl.ds(sid * rpt, rpt)])
    pltpu.sync_copy(col_hbm.at[pl.ds(wid * gpw, gpw)], colv)
    pltpu.sync_copy(row_hbm.at[pl.ds(wid * gpw, gpw)], rowv)
    plsc.subcore_barrier()

    for k in range(K):
      pltpu.async_copy(z_sh.at[colv.at[k]], bufs.at[0, k], sem_g)

    def body(s, carry):
      cur = lax.rem(s, 2)
      for k in range(K):
        pltpu.make_async_copy(z_sh.at[colv.at[0]], bufs.at[cur, k],
                              sem_g).wait()

      @pl.when(s < nsteps - 1)
      def _fire_next():
        for k in range(K):
          pltpu.async_copy(z_sh.at[colv.at[(s + 1) * K + k]],
                           bufs.at[1 - cur, k], sem_g)

      for k in range(K):
        pltpu.async_copy(bufs.at[cur, k], acc.at[rowv.at[s * K + k]],
                         sem_s, add=True)
      for k in range(K):
        pltpu.make_async_copy(bufs.at[cur, k], acc.at[rowv.at[0]],
                              sem_s).wait()
      return carry

    lax.fori_loop(0, nsteps, body, 0)
    plsc.subcore_barrier()
    pltpu.sync_copy(acc.at[pl.ds(sid * rpt, rpt)],
                    out_hbm.at[cid, pl.ds(sid * rpt, rpt)])

  return gs_kernel


def _tc_a0_body(x_ref, w1_ref, b1_ref, hlin0_ref, y1_ref):
  x = x_ref[...]
  hlin0_ref[...] = (
      jnp.dot(x, w1_ref[0], preferred_element_type=jnp.float32) + b1_ref[...])
  y1_ref[...] = jnp.dot(x, w1_ref[1], preferred_element_type=jnp.float32)


def _tc_a1_body(n, deg_ref, y1_ref, dinv_ref, z1p_ref):
  deg = deg_ref[:, 0:1] + deg_ref[:, 1:2]
  dinv = jnp.where(deg > 0, lax.rsqrt(jnp.maximum(deg, 1.0)), 0.0)
  dinv_ref[...] = dinv
  z1p_ref[...] = jnp.zeros_like(z1p_ref)
  z1p_ref[0:n, :] = dinv[0:n, :] * y1_ref[...]


def _tc_b_body(n, hlin0_ref, g1_ref, dinv_ref, w2_ref, b2_ref,
               hlin20_ref, z2p_ref):
  dinv = dinv_ref[0:n, :]
  s = g1_ref[0, 0:n, :] + g1_ref[1, 0:n, :]
  h = jnp.maximum(hlin0_ref[...] - dinv * s, 0.0)
  hlin20_ref[...] = (
      jnp.dot(h, w2_ref[0], preferred_element_type=jnp.float32) + b2_ref[...])
  y2 = jnp.dot(h, w2_ref[1], preferred_element_type=jnp.float32)
  z2p_ref[...] = jnp.zeros_like(z2p_ref)
  z2p_ref[0:n, :] = dinv * y2


def _tc_c_body(n, hlin20_ref, g2_ref, dinv_ref, out_ref):
  dinv = dinv_ref[0:n, :]
  s = g2_ref[0, 0:n, :] + g2_ref[1, 0:n, :]
  o = hlin20_ref[...] - dinv * s
  m = jnp.max(o, axis=1, keepdims=True)
  lse = jnp.log(jnp.sum(jnp.exp(o - m), axis=1, keepdims=True)) + m
  out_ref[...] = o - lse


def kernel(x, edge_index, W1, b1, W2, b2):
  n, f_in = x.shape
  e = edge_index.shape[1]
  hid = W1.shape[2]
  ncls = W2.shape[2]

  n_pad = -(-(n + 1) // (NS * GRP)) * (NS * GRP)
  e_pad = -(-e // (NC * NS * GRP * 8)) * (NC * NS * GRP * 8)
  n_groups = e_pad // GRP

  pad = jnp.full((e_pad - e,), n, dtype=jnp.int32)
  row2d = jnp.concatenate([edge_index[0], pad]).reshape(n_groups, GRP)
  col2d = jnp.concatenate([edge_index[1], pad]).reshape(n_groups, GRP)
  zeros1 = jnp.zeros((n_pad,), jnp.float32)
  zeros_h = jnp.zeros((n_pad, hid), jnp.float32)
  zeros_c = jnp.zeros((n_pad, ncls), jnp.float32)

  deg2 = _make_deg_kernel(n_pad, n_groups)(row2d, zeros1)
  degT = deg2.T

  hlin0, y1 = pl.pallas_call(
      _tc_a0_body,
      out_shape=[
          jax.ShapeDtypeStruct((n, hid), jnp.float32),
          jax.ShapeDtypeStruct((n, hid), jnp.float32),
      ],
  )(x, W1, b1.reshape(1, -1))

  dinvp, z1p = pl.pallas_call(
      functools.partial(_tc_a1_body, n),
      out_shape=[
          jax.ShapeDtypeStruct((n_pad, 1), jnp.float32),
          jax.ShapeDtypeStruct((n_pad, hid), jnp.float32),
      ],
  )(degT, y1)

  g1 = _make_gs_kernel(n_pad, n_groups, hid)(z1p, col2d, row2d, zeros_h)

  hlin20, z2p = pl.pallas_call(
      functools.partial(_tc_b_body, n),
      out_shape=[
          jax.ShapeDtypeStruct((n, ncls), jnp.float32),
          jax.ShapeDtypeStruct((n_pad, ncls), jnp.float32),
      ],
  )(hlin0, g1, dinvp, W2, b2.reshape(1, -1))

  g2 = _make_gs_kernel(n_pad, n_groups, ncls)(z2p, col2d, row2d, zeros_c)

  out = pl.pallas_call(
      functools.partial(_tc_c_body, n),
      out_shape=jax.ShapeDtypeStruct((n, ncls), jnp.float32),
  )(hlin20, g2, dinvp)

  return out


# speedup vs baseline: 47.9244x; 1.0760x over previous
import functools

import jax
import jax.numpy as jnp
from jax import lax
from jax.experimental import pallas as pl
from jax.experimental.pallas import tpu as pltpu
from jax.experimental.pallas import tpu_sc as plsc

NC = 2
NS = 16
GRP = 128


def _mesh():
  return plsc.VectorSubcoreMesh(core_axis_name="c", subcore_axis_name="s")


_SC_PARAMS = pltpu.CompilerParams(use_tc_tiling_on_sc=False)


_NW = NC * NS


def _group_split(n_groups, k):
  base = n_groups // _NW
  rem = n_groups % _NW
  ldg = base + (1 if rem else 0)
  unif = -(-ldg // k) * k
  return base, rem, ldg, unif


def _load_indices(edge_hbm, dim, goff, dst, ldg, base, rem, unif, n_real, n):
  pltpu.sync_copy(edge_hbm.at[dim, pl.ds(goff, ldg)], dst.at[pl.ds(0, ldg)])
  for r in range(base if rem else ldg, unif):
    @pl.when(r >= n_real)
    def _fill():
      for i in range(GRP // 16):
        dst[r, pl.ds(i * 16, 16)] = jnp.full((16,), n, jnp.int32)


@functools.lru_cache(maxsize=None)
def _make_deg_kernel(n_pad, n_groups, n):
  rpt = n_pad // NS
  DK = 8
  base, rem, ldg, unif = _group_split(n_groups, DK)

  @functools.partial(
      pl.kernel, mesh=_mesh(),
      out_type=jax.ShapeDtypeStruct((NC, n_pad), jnp.float32),
      compiler_params=_SC_PARAMS,
      scratch_types=[
          pltpu.VMEM((unif, GRP), jnp.int32),
          pltpu.VMEM((GRP,), jnp.float32),
          pltpu.VMEM_SHARED((n_pad,), jnp.float32),
          pltpu.SemaphoreType.DMA,
      ],
  )
  def deg_kernel(edge_hbm, zeros_hbm, out_hbm, rowv, ones_v, acc, sem):
    cid = lax.axis_index("c")
    sid = lax.axis_index("s")
    wid = sid * NC + cid
    goff = wid * base + (jnp.maximum(0, wid - (_NW - rem)) if rem else 0)
    n_real = base + jnp.where(wid >= _NW - rem, 1, 0) if rem else base
    for i in range(GRP // 16):
      ones_v[pl.ds(i * 16, 16)] = jnp.ones((16,), jnp.float32)
    pltpu.sync_copy(zeros_hbm.at[pl.ds(sid * rpt, rpt)],
                    acc.at[pl.ds(sid * rpt, rpt)])
    plsc.subcore_barrier()
    _load_indices(edge_hbm, 0, goff, rowv, ldg, base, rem, unif, n_real, n)

    def body(s, carry):
      for k in range(DK):
        pltpu.async_copy(ones_v, acc.at[rowv.at[s * DK + k]], sem, add=True)
      for k in range(DK):
        pltpu.make_async_copy(ones_v, acc.at[rowv.at[0]], sem).wait()
      return carry

    lax.fori_loop(0, unif // DK, body, 0)
    plsc.subcore_barrier()
    pltpu.sync_copy(acc.at[pl.ds(sid * rpt, rpt)],
                    out_hbm.at[cid, pl.ds(sid * rpt, rpt)])

  return deg_kernel


@functools.lru_cache(maxsize=None)
def _make_gs_kernel(n_pad, n_groups, n, width):
  rpt = n_pad // NS
  K = 8
  base, rem, ldg, unif = _group_split(n_groups, K)
  nsteps = unif // K

  @functools.partial(
      pl.kernel, mesh=_mesh(),
      out_type=jax.ShapeDtypeStruct((NC, n_pad, width), jnp.float32),
      compiler_params=_SC_PARAMS,
      scratch_types=[
          pltpu.VMEM((unif, GRP), jnp.int32),
          pltpu.VMEM((unif, GRP), jnp.int32),
          pltpu.VMEM((2, K, GRP, width), jnp.float32),
          pltpu.VMEM_SHARED((n_pad, width), jnp.float32),
          pltpu.VMEM_SHARED((n_pad, width), jnp.float32),
          pltpu.SemaphoreType.DMA,
          pltpu.SemaphoreType.DMA,
      ],
  )
  def gs_kernel(z_hbm, edge_hbm, zeros_hbm, out_hbm,
                colv, rowv, bufs, acc, z_sh, sem_g, sem_s):
    cid = lax.axis_index("c")
    sid = lax.axis_index("s")
    wid = sid * NC + cid
    goff = wid * base + (jnp.maximum(0, wid - (_NW - rem)) if rem else 0)
    n_real = base + jnp.where(wid >= _NW - rem, 1, 0) if rem else base
    pltpu.sync_copy(zeros_hbm.at[pl.ds(sid * rpt, rpt)],
                    acc.at[pl.ds(sid * rpt, rpt)])
    pltpu.sync_copy(z_hbm.at[pl.ds(sid * rpt, rpt)],
                    z_sh.at[pl.ds(sid * rpt, rpt)])
    _load_indices(edge_hbm, 1, goff, colv, ldg, base, rem, unif, n_real, n)
    _load_indices(edge_hbm, 0, goff, rowv, ldg, base, rem, unif, n_real, n)
    plsc.subcore_barrier()

    for k in range(K):
      pltpu.async_copy(z_sh.at[colv.at[k]], bufs.at[0, k], sem_g)

    def body(s, carry):
      cur = lax.rem(s, 2)
      for k in range(K):
        pltpu.make_async_copy(z_sh.at[colv.at[0]], bufs.at[cur, k],
                              sem_g).wait()

      @pl.when(s < nsteps - 1)
      def _fire_next():
        for k in range(K):
          pltpu.async_copy(z_sh.at[colv.at[(s + 1) * K + k]],
                           bufs.at[1 - cur, k], sem_g)

      for k in range(K):
        pltpu.async_copy(bufs.at[cur, k], acc.at[rowv.at[s * K + k]],
                         sem_s, add=True)
      for k in range(K):
        pltpu.make_async_copy(bufs.at[cur, k], acc.at[rowv.at[0]],
                              sem_s).wait()
      return carry

    lax.fori_loop(0, nsteps, body, 0)
    plsc.subcore_barrier()
    pltpu.sync_copy(acc.at[pl.ds(sid * rpt, rpt)],
                    out_hbm.at[cid, pl.ds(sid * rpt, rpt)])

  return gs_kernel


def _tc_a0_body(x_ref, w1_ref, b1_ref, hlin0_ref, y1_ref):
  x = x_ref[...]
  hlin0_ref[...] = (
      jnp.dot(x, w1_ref[0], preferred_element_type=jnp.float32) + b1_ref[...])
  y1_ref[...] = jnp.dot(x, w1_ref[1], preferred_element_type=jnp.float32)


def _tc_a1_body(n, deg_ref, y1_ref, dinv_ref, z1p_ref):
  deg = deg_ref[:, 0:1] + deg_ref[:, 1:2]
  dinv = jnp.where(deg > 0, lax.rsqrt(jnp.maximum(deg, 1.0)), 0.0)
  dinv_ref[...] = dinv
  z1p_ref[...] = jnp.zeros_like(z1p_ref)
  z1p_ref[0:n, :] = dinv[0:n, :] * y1_ref[...]


def _tc_b_body(n, hlin0_ref, g1_ref, dinv_ref, w2_ref, b2_ref,
               hlin20_ref, z2p_ref):
  dinv = dinv_ref[0:n, :]
  s = g1_ref[0, 0:n, :] + g1_ref[1, 0:n, :]
  h = jnp.maximum(hlin0_ref[...] - dinv * s, 0.0)
  hlin20_ref[...] = (
      jnp.dot(h, w2_ref[0], preferred_element_type=jnp.float32) + b2_ref[...])
  y2 = jnp.dot(h, w2_ref[1], preferred_element_type=jnp.float32)
  z2p_ref[...] = jnp.zeros_like(z2p_ref)
  z2p_ref[0:n, :] = dinv * y2


def _tc_c_body(n, hlin20_ref, g2_ref, dinv_ref, out_ref):
  dinv = dinv_ref[0:n, :]
  s = g2_ref[0, 0:n, :] + g2_ref[1, 0:n, :]
  o = hlin20_ref[...] - dinv * s
  m = jnp.max(o, axis=1, keepdims=True)
  lse = jnp.log(jnp.sum(jnp.exp(o - m), axis=1, keepdims=True)) + m
  out_ref[...] = o - lse


def kernel(x, edge_index, W1, b1, W2, b2):
  n, f_in = x.shape
  e = edge_index.shape[1]
  hid = W1.shape[2]
  ncls = W2.shape[2]

  n_pad = -(-(n + 1) // (NS * GRP)) * (NS * GRP)
  if e % GRP:
    tail = jnp.full((2, GRP - e % GRP), n, dtype=jnp.int32)
    edge_index = jnp.concatenate([edge_index, tail], axis=1)
  n_groups = edge_index.shape[1] // GRP
  edge3 = edge_index.reshape(2, n_groups, GRP)
  zeros1 = jnp.zeros((n_pad,), jnp.float32)
  zeros_h = jnp.zeros((n_pad, hid), jnp.float32)
  zeros_c = jnp.zeros((n_pad, ncls), jnp.float32)

  deg2 = _make_deg_kernel(n_pad, n_groups, n)(edge3, zeros1)
  degT = deg2.T

  hlin0, y1 = pl.pallas_call(
      _tc_a0_body,
      out_shape=[
          jax.ShapeDtypeStruct((n, hid), jnp.float32),
          jax.ShapeDtypeStruct((n, hid), jnp.float32),
      ],
  )(x, W1, b1.reshape(1, -1))

  dinvp, z1p = pl.pallas_call(
      functools.partial(_tc_a1_body, n),
      out_shape=[
          jax.ShapeDtypeStruct((n_pad, 1), jnp.float32),
          jax.ShapeDtypeStruct((n_pad, hid), jnp.float32),
      ],
  )(degT, y1)

  g1 = _make_gs_kernel(n_pad, n_groups, n, hid)(z1p, edge3, zeros_h)

  hlin20, z2p = pl.pallas_call(
      functools.partial(_tc_b_body, n),
      out_shape=[
          jax.ShapeDtypeStruct((n, ncls), jnp.float32),
          jax.ShapeDtypeStruct((n_pad, ncls), jnp.float32),
      ],
  )(hlin0, g1, dinvp, W2, b2.reshape(1, -1))

  g2 = _make_gs_kernel(n_pad, n_groups, n, ncls)(z2p, edge3, zeros_c)

  out = pl.pallas_call(
      functools.partial(_tc_c_body, n),
      out_shape=jax.ShapeDtypeStruct((n, ncls), jnp.float32),
  )(hlin20, g2, dinvp)

  return out
